# baseline (speedup 1.0000x reference)
import jax
import jax.numpy as jnp
from jax.experimental import pallas as pl
from jax.experimental.pallas import tpu as pltpu


def _pick_tc(T):
    for c in (8, 4, 2, 1):
        if T % c == 0:
            return c
    return 1


def _reorder_gates(w, Hp):
    r = w.reshape(w.shape[:-1] + (4, Hp))
    r = r[..., jnp.array([0, 1, 3, 2]), :]
    return r.reshape(w.shape)


def _block_diag(a, b):
    ra, ca = a.shape
    rb, cb = b.shape
    z1 = jnp.zeros((ra, cb), a.dtype)
    z2 = jnp.zeros((rb, ca), b.dtype)
    return jnp.concatenate(
        [jnp.concatenate([a, z1], axis=1),
         jnp.concatenate([z2, b], axis=1)], axis=0)


def _cell_update(gf, gb, c, Hp):
    sf = jax.nn.sigmoid(gf[:, :3 * Hp])
    tf = jnp.tanh(gf[:, 3 * Hp:])
    sb = jax.nn.sigmoid(gb[:, :3 * Hp])
    tb = jnp.tanh(gb[:, 3 * Hp:])
    cf = sf[:, Hp:2 * Hp] * c[:, :Hp] + sf[:, :Hp] * tf
    cb = sb[:, Hp:2 * Hp] * c[:, Hp:] + sb[:, :Hp] * tb
    hf = sf[:, 2 * Hp:] * jnp.tanh(cf)
    hb = sb[:, 2 * Hp:] * jnp.tanh(cb)
    return hf, hb, cf, cb


def _make_layer0_kernel(Tc, B, Hp):
    G = 4 * Hp

    def body(xf_ref, xb_ref, wf_ref, wb_ref, bf_ref, bb_ref, wbig_ref,
             hf_out, hb_out, h_sc, c_sc):
        @pl.when(pl.program_id(0) == 0)
        def _init():
            h_sc[...] = jnp.zeros_like(h_sc)
            c_sc[...] = jnp.zeros_like(c_sc)

        pf = jnp.dot(xf_ref[...], wf_ref[...],
                     preferred_element_type=jnp.float32) + bf_ref[...]
        pb = jnp.dot(xb_ref[...], wb_ref[...],
                     preferred_element_type=jnp.float32) + bb_ref[...]
        wbig = wbig_ref[...]
        h = h_sc[...]
        c = c_sc[...]
        for s in range(Tc):
            gd = jnp.dot(h, wbig, preferred_element_type=jnp.float32)
            gf = gd[:, :G] + pf[s * B:(s + 1) * B]
            gb = gd[:, G:] + pb[(Tc - 1 - s) * B:(Tc - s) * B]
            hf, hb, cf, cb = _cell_update(gf, gb, c, Hp)
            hf_out[s * B:(s + 1) * B, :] = hf
            hb_out[(Tc - 1 - s) * B:(Tc - s) * B, :] = hb
            h = jnp.concatenate([hf, hb], axis=1)
            c = jnp.concatenate([cf, cb], axis=1)
        h_sc[...] = h
        c_sc[...] = c

    return body


def _make_layer1_kernel(Tc, B, Hp, nT):
    G = 4 * Hp

    def body(hf_t, hb_t, hf_r, hb_r, w1f_ref, w1b_ref, b1f_ref, b1b_ref,
             wbig_ref, fcwf_ref, fcwb_ref, fcb_ref, out_ref,
             h_sc, c_sc, head_sc):
        t = pl.program_id(0)

        @pl.when(t == 0)
        def _init():
            h_sc[...] = jnp.zeros_like(h_sc)
            c_sc[...] = jnp.zeros_like(c_sc)

        catf = jnp.concatenate([hf_t[...], hb_t[...]], axis=1)
        catb = jnp.concatenate([hf_r[...], hb_r[...]], axis=1)
        pf = jnp.dot(catf, w1f_ref[...],
                     preferred_element_type=jnp.float32) + b1f_ref[...]
        pb = jnp.dot(catb, w1b_ref[...],
                     preferred_element_type=jnp.float32) + b1b_ref[...]
        wbig = wbig_ref[...]
        h = h_sc[...]
        c = c_sc[...]
        hb_first = None
        for s in range(Tc):
            gd = jnp.dot(h, wbig, preferred_element_type=jnp.float32)
            gf = gd[:, :G] + pf[s * B:(s + 1) * B]
            gb = gd[:, G:] + pb[(Tc - 1 - s) * B:(Tc - s) * B]
            hf, hb, cf, cb = _cell_update(gf, gb, c, Hp)
            if s == 0:
                hb_first = hb
            h = jnp.concatenate([hf, hb], axis=1)
            c = jnp.concatenate([cf, cb], axis=1)
        h_sc[...] = h
        c_sc[...] = c

        @pl.when(t == 0)
        def _store_bwd_head():
            head_sc[...] = jnp.dot(
                hb_first, fcwb_ref[...],
                preferred_element_type=jnp.float32) + fcb_ref[...]

        @pl.when(t == nT - 1)
        def _finalize():
            logits = head_sc[...] + jnp.dot(
                h[:, :Hp], fcwf_ref[...], preferred_element_type=jnp.float32)
            m = jnp.max(logits, axis=-1, keepdims=True)
            shifted = logits - m
            lse = jnp.log(jnp.sum(jnp.exp(shifted), axis=-1, keepdims=True))
            out_ref[...] = shifted - lse

    return body


def kernel(embedding, l0_w_in_f0, l0_w_in_b0, l0_b_f, l0_b_b, l0_whh_f,
           l0_whh_b, l1_w_in_f0, l1_w_in_f1, l1_w_in_b0, l1_w_in_b1, l1_b_f,
           l1_b_b, l1_whh_f, l1_whh_b, fc_wf, fc_wb, fc_b, tokens):
    T, B = tokens.shape
    E = embedding.shape[1]
    Hp = l0_whh_f.shape[0]
    G = 4 * Hp
    O = fc_wf.shape[1]
    Tc = _pick_tc(T)
    nT = T // Tc
    RB = Tc * B

    x = jnp.take(embedding, tokens.reshape(-1), axis=0)

    ro = lambda w: _reorder_gates(w, Hp)
    w0f, w0b = ro(l0_w_in_f0), ro(l0_w_in_b0)
    b0f, b0b = ro(l0_b_f), ro(l0_b_b)
    wbig0 = _block_diag(ro(l0_whh_f), ro(l0_whh_b))
    w1f = jnp.concatenate([ro(l1_w_in_f0), ro(l1_w_in_f1)], axis=0)
    w1b = jnp.concatenate([ro(l1_w_in_b0), ro(l1_w_in_b1)], axis=0)
    b1f, b1b = ro(l1_b_f), ro(l1_b_b)
    wbig1 = _block_diag(ro(l1_whh_f), ro(l1_whh_b))

    hf, hb = pl.pallas_call(
        _make_layer0_kernel(Tc, B, Hp),
        out_shape=(jax.ShapeDtypeStruct((T * B, Hp), jnp.float32),
                   jax.ShapeDtypeStruct((T * B, Hp), jnp.float32)),
        grid_spec=pltpu.PrefetchScalarGridSpec(
            num_scalar_prefetch=0,
            grid=(nT,),
            in_specs=[
                pl.BlockSpec((RB, E), lambda t: (t, 0)),
                pl.BlockSpec((RB, E), lambda t: (nT - 1 - t, 0)),
                pl.BlockSpec((E, G), lambda t: (0, 0)),
                pl.BlockSpec((E, G), lambda t: (0, 0)),
                pl.BlockSpec((1, G), lambda t: (0, 0)),
                pl.BlockSpec((1, G), lambda t: (0, 0)),
                pl.BlockSpec((2 * Hp, 2 * G), lambda t: (0, 0)),
            ],
            out_specs=[
                pl.BlockSpec((RB, Hp), lambda t: (t, 0)),
                pl.BlockSpec((RB, Hp), lambda t: (nT - 1 - t, 0)),
            ],
            scratch_shapes=[pltpu.VMEM((B, 2 * Hp), jnp.float32)] * 2,
        ),
        compiler_params=pltpu.CompilerParams(
            dimension_semantics=("arbitrary",)),
    )(x, x, w0f, w0b, b0f, b0b, wbig0)

    out = pl.pallas_call(
        _make_layer1_kernel(Tc, B, Hp, nT),
        out_shape=jax.ShapeDtypeStruct((B, O), jnp.float32),
        grid_spec=pltpu.PrefetchScalarGridSpec(
            num_scalar_prefetch=0,
            grid=(nT,),
            in_specs=[
                pl.BlockSpec((RB, Hp), lambda t: (t, 0)),
                pl.BlockSpec((RB, Hp), lambda t: (t, 0)),
                pl.BlockSpec((RB, Hp), lambda t: (nT - 1 - t, 0)),
                pl.BlockSpec((RB, Hp), lambda t: (nT - 1 - t, 0)),
                pl.BlockSpec((2 * Hp, G), lambda t: (0, 0)),
                pl.BlockSpec((2 * Hp, G), lambda t: (0, 0)),
                pl.BlockSpec((1, G), lambda t: (0, 0)),
                pl.BlockSpec((1, G), lambda t: (0, 0)),
                pl.BlockSpec((2 * Hp, 2 * G), lambda t: (0, 0)),
                pl.BlockSpec((Hp, O), lambda t: (0, 0)),
                pl.BlockSpec((Hp, O), lambda t: (0, 0)),
                pl.BlockSpec((1, O), lambda t: (0, 0)),
            ],
            out_specs=pl.BlockSpec((B, O), lambda t: (0, 0)),
            scratch_shapes=[pltpu.VMEM((B, 2 * Hp), jnp.float32),
                            pltpu.VMEM((B, 2 * Hp), jnp.float32),
                            pltpu.VMEM((B, O), jnp.float32)],
        ),
        compiler_params=pltpu.CompilerParams(
            dimension_semantics=("arbitrary",)),
    )(hf, hb, hf, hb, w1f, w1b, b1f, b1b, wbig1, fc_wf, fc_wb, fc_b)

    return out


# speedup vs baseline: 1.2746x; 1.2746x over previous
import jax
import jax.numpy as jnp
from jax.experimental import pallas as pl
from jax.experimental.pallas import tpu as pltpu


def _pick_tc(T):
    for c in (8, 4, 2, 1):
        if T % c == 0:
            return c
    return 1


def _gate_mult(B, Hp, G2):
    lane = jax.lax.broadcasted_iota(jnp.int32, (B, G2), 1)
    grp = (lane % (4 * Hp)) // Hp
    return jnp.where(grp == 2, 1.0, 0.5).astype(jnp.float32)


def _dual_cell(th, c, Hp, G):
    i_f = 0.5 + 0.5 * th[:, 0 * Hp:1 * Hp]
    f_f = 0.5 + 0.5 * th[:, 1 * Hp:2 * Hp]
    g_f = th[:, 2 * Hp:3 * Hp]
    o_f = 0.5 + 0.5 * th[:, 3 * Hp:4 * Hp]
    i_b = 0.5 + 0.5 * th[:, G + 0 * Hp:G + 1 * Hp]
    f_b = 0.5 + 0.5 * th[:, G + 1 * Hp:G + 2 * Hp]
    g_b = th[:, G + 2 * Hp:G + 3 * Hp]
    o_b = 0.5 + 0.5 * th[:, G + 3 * Hp:G + 4 * Hp]
    cf = f_f * c[:, :Hp] + i_f * g_f
    cb = f_b * c[:, Hp:] + i_b * g_b
    hf = o_f * jnp.tanh(cf)
    hb = o_b * jnp.tanh(cb)
    return hf, hb, cf, cb


def _make_layer0_kernel(Tc, B, Hp):
    G = 4 * Hp

    def body(xf_ref, xb_ref, wf_ref, wb_ref, bf_ref, bb_ref,
             whhf_ref, whhb_ref, hf_out, hb_out, h_sc, c_sc, wbig_sc):
        @pl.when(pl.program_id(0) == 0)
        def _init():
            h_sc[...] = jnp.zeros_like(h_sc)
            c_sc[...] = jnp.zeros_like(c_sc)
            wbig_sc[...] = jnp.zeros_like(wbig_sc)
            wbig_sc[:Hp, :G] = whhf_ref[...]
            wbig_sc[Hp:, G:] = whhb_ref[...]

        pf = jnp.dot(xf_ref[...], wf_ref[...],
                     preferred_element_type=jnp.float32) + bf_ref[...]
        pb = jnp.dot(xb_ref[...], wb_ref[...],
                     preferred_element_type=jnp.float32) + bb_ref[...]
        wbig = wbig_sc[...]
        mult = _gate_mult(B, Hp, 2 * G)
        h = h_sc[...]
        c = c_sc[...]
        for s in range(Tc):
            gd = jnp.dot(h, wbig, preferred_element_type=jnp.float32)
            pcat = jnp.concatenate(
                [pf[s * B:(s + 1) * B], pb[(Tc - 1 - s) * B:(Tc - s) * B]],
                axis=1)
            th = jnp.tanh((gd + pcat) * mult)
            hf, hb, cf, cb = _dual_cell(th, c, Hp, G)
            hf_out[s * B:(s + 1) * B, :] = hf
            hb_out[(Tc - 1 - s) * B:(Tc - s) * B, :] = hb
            h = jnp.concatenate([hf, hb], axis=1)
            c = jnp.concatenate([cf, cb], axis=1)
        h_sc[...] = h
        c_sc[...] = c

    return body


def _make_layer1_kernel(Tc, B, Hp, nT):
    G = 4 * Hp

    def body(hf_t, hb_t, hf_r, hb_r, w1f0_ref, w1f1_ref, w1b0_ref, w1b1_ref,
             b1f_ref, b1b_ref, whhf_ref, whhb_ref,
             fcwf_ref, fcwb_ref, fcb_ref, out_ref,
             h_sc, c_sc, head_sc, wbig_sc, w1f_sc, w1b_sc):
        t = pl.program_id(0)

        @pl.when(t == 0)
        def _init():
            h_sc[...] = jnp.zeros_like(h_sc)
            c_sc[...] = jnp.zeros_like(c_sc)
            wbig_sc[...] = jnp.zeros_like(wbig_sc)
            wbig_sc[:Hp, :G] = whhf_ref[...]
            wbig_sc[Hp:, G:] = whhb_ref[...]
            w1f_sc[:Hp, :] = w1f0_ref[...]
            w1f_sc[Hp:, :] = w1f1_ref[...]
            w1b_sc[:Hp, :] = w1b0_ref[...]
            w1b_sc[Hp:, :] = w1b1_ref[...]

        catf = jnp.concatenate([hf_t[...], hb_t[...]], axis=1)
        catb = jnp.concatenate([hf_r[...], hb_r[...]], axis=1)
        pf = jnp.dot(catf, w1f_sc[...],
                     preferred_element_type=jnp.float32) + b1f_ref[...]
        pb = jnp.dot(catb, w1b_sc[...],
                     preferred_element_type=jnp.float32) + b1b_ref[...]
        wbig = wbig_sc[...]
        mult = _gate_mult(B, Hp, 2 * G)
        h = h_sc[...]
        c = c_sc[...]
        hb_first = None
        for s in range(Tc):
            gd = jnp.dot(h, wbig, preferred_element_type=jnp.float32)
            pcat = jnp.concatenate(
                [pf[s * B:(s + 1) * B], pb[(Tc - 1 - s) * B:(Tc - s) * B]],
                axis=1)
            th = jnp.tanh((gd + pcat) * mult)
            hf, hb, cf, cb = _dual_cell(th, c, Hp, G)
            if s == 0:
                hb_first = hb
            h = jnp.concatenate([hf, hb], axis=1)
            c = jnp.concatenate([cf, cb], axis=1)
        h_sc[...] = h
        c_sc[...] = c

        @pl.when(t == 0)
        def _store_bwd_head():
            head_sc[...] = jnp.dot(
                hb_first, fcwb_ref[...],
                preferred_element_type=jnp.float32) + fcb_ref[...]

        @pl.when(t == nT - 1)
        def _finalize():
            logits = head_sc[...] + jnp.dot(
                h[:, :Hp], fcwf_ref[...], preferred_element_type=jnp.float32)
            m = jnp.max(logits, axis=-1, keepdims=True)
            shifted = logits - m
            lse = jnp.log(jnp.sum(jnp.exp(shifted), axis=-1, keepdims=True))
            out_ref[...] = shifted - lse

    return body


def kernel(embedding, l0_w_in_f0, l0_w_in_b0, l0_b_f, l0_b_b, l0_whh_f,
           l0_whh_b, l1_w_in_f0, l1_w_in_f1, l1_w_in_b0, l1_w_in_b1, l1_b_f,
           l1_b_b, l1_whh_f, l1_whh_b, fc_wf, fc_wb, fc_b, tokens):
    T, B = tokens.shape
    E = embedding.shape[1]
    Hp = l0_whh_f.shape[0]
    G = 4 * Hp
    O = fc_wf.shape[1]
    Tc = _pick_tc(T)
    nT = T // Tc
    RB = Tc * B

    x = jnp.take(embedding, tokens.reshape(-1), axis=0)

    hf, hb = pl.pallas_call(
        _make_layer0_kernel(Tc, B, Hp),
        out_shape=(jax.ShapeDtypeStruct((T * B, Hp), jnp.float32),
                   jax.ShapeDtypeStruct((T * B, Hp), jnp.float32)),
        grid_spec=pltpu.PrefetchScalarGridSpec(
            num_scalar_prefetch=0,
            grid=(nT,),
            in_specs=[
                pl.BlockSpec((RB, E), lambda t: (t, 0)),
                pl.BlockSpec((RB, E), lambda t: (nT - 1 - t, 0)),
                pl.BlockSpec((E, G), lambda t: (0, 0)),
                pl.BlockSpec((E, G), lambda t: (0, 0)),
                pl.BlockSpec((1, G), lambda t: (0, 0)),
                pl.BlockSpec((1, G), lambda t: (0, 0)),
                pl.BlockSpec((Hp, G), lambda t: (0, 0)),
                pl.BlockSpec((Hp, G), lambda t: (0, 0)),
            ],
            out_specs=[
                pl.BlockSpec((RB, Hp), lambda t: (t, 0)),
                pl.BlockSpec((RB, Hp), lambda t: (nT - 1 - t, 0)),
            ],
            scratch_shapes=[pltpu.VMEM((B, 2 * Hp), jnp.float32),
                            pltpu.VMEM((B, 2 * Hp), jnp.float32),
                            pltpu.VMEM((2 * Hp, 2 * G), jnp.float32)],
        ),
        compiler_params=pltpu.CompilerParams(
            dimension_semantics=("arbitrary",)),
    )(x, x, l0_w_in_f0, l0_w_in_b0, l0_b_f, l0_b_b, l0_whh_f, l0_whh_b)

    out = pl.pallas_call(
        _make_layer1_kernel(Tc, B, Hp, nT),
        out_shape=jax.ShapeDtypeStruct((B, O), jnp.float32),
        grid_spec=pltpu.PrefetchScalarGridSpec(
            num_scalar_prefetch=0,
            grid=(nT,),
            in_specs=[
                pl.BlockSpec((RB, Hp), lambda t: (t, 0)),
                pl.BlockSpec((RB, Hp), lambda t: (t, 0)),
                pl.BlockSpec((RB, Hp), lambda t: (nT - 1 - t, 0)),
                pl.BlockSpec((RB, Hp), lambda t: (nT - 1 - t, 0)),
                pl.BlockSpec((Hp, G), lambda t: (0, 0)),
                pl.BlockSpec((Hp, G), lambda t: (0, 0)),
                pl.BlockSpec((Hp, G), lambda t: (0, 0)),
                pl.BlockSpec((Hp, G), lambda t: (0, 0)),
                pl.BlockSpec((1, G), lambda t: (0, 0)),
                pl.BlockSpec((1, G), lambda t: (0, 0)),
                pl.BlockSpec((Hp, G), lambda t: (0, 0)),
                pl.BlockSpec((Hp, G), lambda t: (0, 0)),
                pl.BlockSpec((Hp, O), lambda t: (0, 0)),
                pl.BlockSpec((Hp, O), lambda t: (0, 0)),
                pl.BlockSpec((1, O), lambda t: (0, 0)),
            ],
            out_specs=pl.BlockSpec((B, O), lambda t: (0, 0)),
            scratch_shapes=[pltpu.VMEM((B, 2 * Hp), jnp.float32),
                            pltpu.VMEM((B, 2 * Hp), jnp.float32),
                            pltpu.VMEM((B, O), jnp.float32),
                            pltpu.VMEM((2 * Hp, 2 * G), jnp.float32),
                            pltpu.VMEM((2 * Hp, G), jnp.float32),
                            pltpu.VMEM((2 * Hp, G), jnp.float32)],
        ),
        compiler_params=pltpu.CompilerParams(
            dimension_semantics=("arbitrary",)),
    )(hf, hb, hf, hb, l1_w_in_f0, l1_w_in_f1, l1_w_in_b0, l1_w_in_b1,
      l1_b_f, l1_b_b, l1_whh_f, l1_whh_b, fc_wf, fc_wb, fc_b)

    return out


# speedup vs baseline: 1.3087x; 1.0268x over previous
import jax
import jax.numpy as jnp
from jax.experimental import pallas as pl
from jax.experimental.pallas import tpu as pltpu


def _pick_tc(T):
    for c in (8, 4, 2, 1):
        if T % c == 0:
            return c
    return 1


def _gate_mult(B, Hp, G2):
    lane = jax.lax.broadcasted_iota(jnp.int32, (B, G2), 1)
    grp = (lane % (4 * Hp)) // Hp
    return jnp.where(grp == 2, 1.0, 0.5).astype(jnp.float32)


def _dual_cell(th, c, Hp, G):
    i_f = 0.5 + 0.5 * th[:, 0 * Hp:1 * Hp]
    f_f = 0.5 + 0.5 * th[:, 1 * Hp:2 * Hp]
    g_f = th[:, 2 * Hp:3 * Hp]
    o_f = 0.5 + 0.5 * th[:, 3 * Hp:4 * Hp]
    i_b = 0.5 + 0.5 * th[:, G + 0 * Hp:G + 1 * Hp]
    f_b = 0.5 + 0.5 * th[:, G + 1 * Hp:G + 2 * Hp]
    g_b = th[:, G + 2 * Hp:G + 3 * Hp]
    o_b = 0.5 + 0.5 * th[:, G + 3 * Hp:G + 4 * Hp]
    cf = f_f * c[:, :Hp] + i_f * g_f
    cb = f_b * c[:, Hp:] + i_b * g_b
    hf = o_f * jnp.tanh(cf)
    hb = o_b * jnp.tanh(cb)
    return hf, hb, cf, cb


def _make_fused_kernel(Tc, B, Hp, nT):
    G = 4 * Hp
    RB = Tc * B

    def body(xf_ref, xb_ref, w0f_ref, w0b_ref, b0f_ref, b0b_ref,
             whh0f_ref, whh0b_ref,
             w1f0_ref, w1f1_ref, w1b0_ref, w1b1_ref, b1f_ref, b1b_ref,
             whh1f_ref, whh1b_ref, fcwf_ref, fcwb_ref, fcb_ref,
             out_ref,
             h_sc, c_sc, hfseq_sc, hbseq_sc, head_sc,
             wbig0_sc, wbig1_sc, w1f_sc, w1b_sc):
        p = pl.program_id(0)
        t = pl.program_id(1)

        @pl.when((p == 0) & (t == 0))
        def _build_weights():
            wbig0_sc[...] = jnp.zeros_like(wbig0_sc)
            wbig0_sc[:Hp, :G] = whh0f_ref[...]
            wbig0_sc[Hp:, G:] = whh0b_ref[...]
            wbig1_sc[...] = jnp.zeros_like(wbig1_sc)
            wbig1_sc[:Hp, :G] = whh1f_ref[...]
            wbig1_sc[Hp:, G:] = whh1b_ref[...]
            w1f_sc[:Hp, :] = w1f0_ref[...]
            w1f_sc[Hp:, :] = w1f1_ref[...]
            w1b_sc[:Hp, :] = w1b0_ref[...]
            w1b_sc[Hp:, :] = w1b1_ref[...]

        @pl.when(t == 0)
        def _reinit_state():
            h_sc[...] = jnp.zeros_like(h_sc)
            c_sc[...] = jnp.zeros_like(c_sc)

        mult = _gate_mult(B, Hp, 2 * G)

        @pl.when(p == 0)
        def _layer0():
            pf = jnp.dot(xf_ref[...], w0f_ref[...],
                         preferred_element_type=jnp.float32) + b0f_ref[...]
            pb = jnp.dot(xb_ref[...], w0b_ref[...],
                         preferred_element_type=jnp.float32) + b0b_ref[...]
            wbig = wbig0_sc[...]
            h = h_sc[...]
            c = c_sc[...]
            for s in range(Tc):
                gd = jnp.dot(h, wbig, preferred_element_type=jnp.float32)
                pcat = jnp.concatenate(
                    [pf[s * B:(s + 1) * B],
                     pb[(Tc - 1 - s) * B:(Tc - s) * B]], axis=1)
                th = jnp.tanh((gd + pcat) * mult)
                hf, hb, cf, cb = _dual_cell(th, c, Hp, G)
                hfseq_sc[pl.ds(t * RB + s * B, B), :] = hf
                hbseq_sc[pl.ds((nT - 1 - t) * RB + (Tc - 1 - s) * B, B), :] = hb
                h = jnp.concatenate([hf, hb], axis=1)
                c = jnp.concatenate([cf, cb], axis=1)
            h_sc[...] = h
            c_sc[...] = c

        @pl.when(p == 1)
        def _layer1():
            hf_t = hfseq_sc[pl.ds(t * RB, RB), :]
            hb_t = hbseq_sc[pl.ds(t * RB, RB), :]
            hf_r = hfseq_sc[pl.ds((nT - 1 - t) * RB, RB), :]
            hb_r = hbseq_sc[pl.ds((nT - 1 - t) * RB, RB), :]
            catf = jnp.concatenate([hf_t, hb_t], axis=1)
            catb = jnp.concatenate([hf_r, hb_r], axis=1)
            pf = jnp.dot(catf, w1f_sc[...],
                         preferred_element_type=jnp.float32) + b1f_ref[...]
            pb = jnp.dot(catb, w1b_sc[...],
                         preferred_element_type=jnp.float32) + b1b_ref[...]
            wbig = wbig1_sc[...]
            h = h_sc[...]
            c = c_sc[...]
            hb_first = None
            for s in range(Tc):
                gd = jnp.dot(h, wbig, preferred_element_type=jnp.float32)
                pcat = jnp.concatenate(
                    [pf[s * B:(s + 1) * B],
                     pb[(Tc - 1 - s) * B:(Tc - s) * B]], axis=1)
                th = jnp.tanh((gd + pcat) * mult)
                hf, hb, cf, cb = _dual_cell(th, c, Hp, G)
                if s == 0:
                    hb_first = hb
                h = jnp.concatenate([hf, hb], axis=1)
                c = jnp.concatenate([cf, cb], axis=1)
            h_sc[...] = h
            c_sc[...] = c

            @pl.when(t == 0)
            def _store_bwd_head():
                head_sc[...] = jnp.dot(
                    hb_first, fcwb_ref[...],
                    preferred_element_type=jnp.float32) + fcb_ref[...]

            @pl.when(t == nT - 1)
            def _finalize():
                logits = head_sc[...] + jnp.dot(
                    h[:, :Hp], fcwf_ref[...],
                    preferred_element_type=jnp.float32)
                m = jnp.max(logits, axis=-1, keepdims=True)
                shifted = logits - m
                lse = jnp.log(
                    jnp.sum(jnp.exp(shifted), axis=-1, keepdims=True))
                out_ref[...] = shifted - lse

    return body


def kernel(embedding, l0_w_in_f0, l0_w_in_b0, l0_b_f, l0_b_b, l0_whh_f,
           l0_whh_b, l1_w_in_f0, l1_w_in_f1, l1_w_in_b0, l1_w_in_b1, l1_b_f,
           l1_b_b, l1_whh_f, l1_whh_b, fc_wf, fc_wb, fc_b, tokens):
    T, B = tokens.shape
    E = embedding.shape[1]
    Hp = l0_whh_f.shape[0]
    G = 4 * Hp
    O = fc_wf.shape[1]
    Tc = _pick_tc(T)
    nT = T // Tc
    RB = Tc * B

    x = jnp.take(embedding, tokens.reshape(-1), axis=0)

    const = lambda p, t: (0, 0)

    out = pl.pallas_call(
        _make_fused_kernel(Tc, B, Hp, nT),
        out_shape=jax.ShapeDtypeStruct((B, O), jnp.float32),
        grid_spec=pltpu.PrefetchScalarGridSpec(
            num_scalar_prefetch=0,
            grid=(2, nT),
            in_specs=[
                pl.BlockSpec((RB, E),
                             lambda p, t: (jnp.where(p == 0, t, 0), 0)),
                pl.BlockSpec((RB, E),
                             lambda p, t: (jnp.where(p == 0, nT - 1 - t, 0),
                                           0)),
                pl.BlockSpec((E, G), const),
                pl.BlockSpec((E, G), const),
                pl.BlockSpec((1, G), const),
                pl.BlockSpec((1, G), const),
                pl.BlockSpec((Hp, G), const),
                pl.BlockSpec((Hp, G), const),
                pl.BlockSpec((Hp, G), const),
                pl.BlockSpec((Hp, G), const),
                pl.BlockSpec((Hp, G), const),
                pl.BlockSpec((Hp, G), const),
                pl.BlockSpec((1, G), const),
                pl.BlockSpec((1, G), const),
                pl.BlockSpec((Hp, G), const),
                pl.BlockSpec((Hp, G), const),
                pl.BlockSpec((Hp, O), const),
                pl.BlockSpec((Hp, O), const),
                pl.BlockSpec((1, O), const),
            ],
            out_specs=pl.BlockSpec((B, O), const),
            scratch_shapes=[
                pltpu.VMEM((B, 2 * Hp), jnp.float32),
                pltpu.VMEM((B, 2 * Hp), jnp.float32),
                pltpu.VMEM((T * B, Hp), jnp.float32),
                pltpu.VMEM((T * B, Hp), jnp.float32),
                pltpu.VMEM((B, O), jnp.float32),
                pltpu.VMEM((2 * Hp, 2 * G), jnp.float32),
                pltpu.VMEM((2 * Hp, 2 * G), jnp.float32),
                pltpu.VMEM((2 * Hp, G), jnp.float32),
                pltpu.VMEM((2 * Hp, G), jnp.float32),
            ],
        ),
        compiler_params=pltpu.CompilerParams(
            dimension_semantics=("arbitrary", "arbitrary")),
    )(x, x, l0_w_in_f0, l0_w_in_b0, l0_b_f, l0_b_b, l0_whh_f, l0_whh_b,
      l1_w_in_f0, l1_w_in_f1, l1_w_in_b0, l1_w_in_b1, l1_b_f, l1_b_b,
      l1_whh_f, l1_whh_b, fc_wf, fc_wb, fc_b)

    return out
